# baseline (speedup 1.0000x reference)
import jax
import jax.numpy as jnp
from jax.experimental import pallas as pl
from jax.experimental.pallas import tpu as pltpu

_TBL_H = 15
_TBL_W = 15
_BLOCK = 128


def _add_bias_kernel(i0_ref, i1_ref, tbl_ref, x_ref, o_ref, bias_ref):
    @pl.when(pl.program_id(0) == 0)
    def _build_bias():
        kflat = i0_ref[...] * _TBL_W + i1_ref[...]

        def body(t, acc):
            v = tbl_ref[t // _TBL_W, t % _TBL_W]
            return acc + jnp.where(kflat == t, v, 0.0)

        bias_ref[...] = jax.lax.fori_loop(
            0, _TBL_H * _TBL_W, body, jnp.zeros((32, 128), jnp.float32)
        )

    o_ref[...] = x_ref[...] + bias_ref[...][None, :, :]


def kernel(x, pos_embeddings, relative_indices):
    n = x.shape[0] * x.shape[1]
    x3 = x.reshape(n, 32, 128)
    i0 = relative_indices[:, :, 0].reshape(32, 128)
    i1 = relative_indices[:, :, 1].reshape(32, 128)

    grid = (n // _BLOCK,)
    out = pl.pallas_call(
        _add_bias_kernel,
        grid=grid,
        in_specs=[
            pl.BlockSpec((32, 128), lambda i: (0, 0)),
            pl.BlockSpec((32, 128), lambda i: (0, 0)),
            pl.BlockSpec(memory_space=pltpu.SMEM),
            pl.BlockSpec((_BLOCK, 32, 128), lambda i: (i, 0, 0)),
        ],
        out_specs=pl.BlockSpec((_BLOCK, 32, 128), lambda i: (i, 0, 0)),
        out_shape=jax.ShapeDtypeStruct((n, 32, 128), jnp.float32),
        scratch_shapes=[pltpu.VMEM((32, 128), jnp.float32)],
        compiler_params=pltpu.CompilerParams(
            dimension_semantics=("arbitrary",),
        ),
    )(i0, i1, pos_embeddings, x3)
    return out.reshape(x.shape)


# speedup vs baseline: 1.2369x; 1.2369x over previous
import jax
import jax.numpy as jnp
from jax.experimental import pallas as pl
from jax.experimental.pallas import tpu as pltpu

_TBL_H = 15
_TBL_W = 15
_BLOCK = 8


def _add_bias_kernel(i0_ref, i1_ref, tbl_ref, x_ref, o_ref, bias_ref):
    @pl.when(pl.program_id(0) == 0)
    def _build_bias():
        kflat = i0_ref[...] * _TBL_W + i1_ref[...]

        def body(t, acc):
            v = tbl_ref[t // _TBL_W, t % _TBL_W]
            return acc + jnp.where(kflat == t, v, 0.0)

        bias_ref[...] = jax.lax.fori_loop(
            0, _TBL_H * _TBL_W, body, jnp.zeros((64, 64), jnp.float32)
        )

    o_ref[...] = x_ref[...] + bias_ref[...][None, None, :, :]


def kernel(x, pos_embeddings, relative_indices):
    nb, nh, h, w = x.shape
    i0 = relative_indices[:, :, 0]
    i1 = relative_indices[:, :, 1]

    out = pl.pallas_call(
        _add_bias_kernel,
        grid=(nb // _BLOCK,),
        in_specs=[
            pl.BlockSpec((h, w), lambda i: (0, 0)),
            pl.BlockSpec((h, w), lambda i: (0, 0)),
            pl.BlockSpec(memory_space=pltpu.SMEM),
            pl.BlockSpec((_BLOCK, nh, h, w), lambda i: (i, 0, 0, 0)),
        ],
        out_specs=pl.BlockSpec((_BLOCK, nh, h, w), lambda i: (i, 0, 0, 0)),
        out_shape=jax.ShapeDtypeStruct(x.shape, jnp.float32),
        scratch_shapes=[pltpu.VMEM((h, w), jnp.float32)],
        compiler_params=pltpu.CompilerParams(
            dimension_semantics=("arbitrary",),
        ),
    )(i0, i1, pos_embeddings, x)
    return out


# speedup vs baseline: 1.2375x; 1.0005x over previous
import jax
import jax.numpy as jnp
from jax.experimental import pallas as pl
from jax.experimental.pallas import tpu as pltpu

_TBL_H = 15
_TBL_W = 15
_BLOCK = 16


def _gather_bias_kernel(i0_ref, i1_ref, tbl_ref, bias_ref):
    kflat = i0_ref[...] * _TBL_W + i1_ref[...]

    def body(t, acc):
        v = tbl_ref[t // _TBL_W, t % _TBL_W]
        return acc + jnp.where(kflat == t, v, 0.0)

    bias_ref[...] = jax.lax.fori_loop(
        0, _TBL_H * _TBL_W, body, jnp.zeros((64, 64), jnp.float32)
    )


def _add_kernel(bias_ref, x_ref, o_ref):
    o_ref[...] = x_ref[...] + bias_ref[...][None, None, :, :]


def kernel(x, pos_embeddings, relative_indices):
    nb, nh, h, w = x.shape
    i0 = relative_indices[:, :, 0]
    i1 = relative_indices[:, :, 1]

    bias = pl.pallas_call(
        _gather_bias_kernel,
        in_specs=[
            pl.BlockSpec((h, w), lambda: (0, 0)),
            pl.BlockSpec((h, w), lambda: (0, 0)),
            pl.BlockSpec(memory_space=pltpu.SMEM),
        ],
        out_specs=pl.BlockSpec((h, w), lambda: (0, 0)),
        out_shape=jax.ShapeDtypeStruct((h, w), jnp.float32),
    )(i0, i1, pos_embeddings)

    out = pl.pallas_call(
        _add_kernel,
        grid=(nb // _BLOCK,),
        in_specs=[
            pl.BlockSpec((h, w), lambda i: (0, 0)),
            pl.BlockSpec((_BLOCK, nh, h, w), lambda i: (i, 0, 0, 0)),
        ],
        out_specs=pl.BlockSpec((_BLOCK, nh, h, w), lambda i: (i, 0, 0, 0)),
        out_shape=jax.ShapeDtypeStruct(x.shape, jnp.float32),
        compiler_params=pltpu.CompilerParams(
            dimension_semantics=("parallel",),
        ),
    )(bias, x)
    return out
